# baseline (speedup 1.0000x reference)
import functools

import jax
import jax.numpy as jnp
from jax import lax
from jax.experimental import pallas as pl
from jax.experimental.pallas import tpu as pltpu
from jax.experimental.pallas import tpu_sc as plsc

_RES = 7
_NPIX = _RES * _RES
_SLOTS = _NPIX * 16
_CHUNK_PIX = 7
_CHUNK = _CHUNK_PIX * 16
_NCHUNK = _NPIX // _CHUNK_PIX
_SLOTS_PAD = 896
_C = 256
_NROI = 1024
_BR = 128

_WLS = (256, 128, 64, 32)
_BASES = (0, 2 * 256 * 256, 2 * 256 * 256 + 2 * 128 * 128,
          2 * 256 * 256 + 2 * 128 * 128 + 2 * 64 * 64)
_TROWS = _BASES[3] + 2 * 32 * 32
_SCALES = (0.25, 0.125, 0.0625, 0.03125)


def _prep_body(x1_ref, y1_ref, x2_ref, y2_ref, idx_ref, w_ref):
    i32 = jnp.int32
    f32 = jnp.float32
    pid = pl.program_id(0)
    x1 = x1_ref[...]
    y1 = y1_ref[...]
    x2 = x2_ref[...]
    y2 = y2_ref[...]

    area = (x2 - x1 + 1.0) * (y2 - y1 + 1.0)
    size = jnp.sqrt(area)
    lvlf = jnp.floor(4.0 + jnp.log2(size / 224.0 + 1e-6))
    lvl = jnp.clip(lvlf, 2.0, 5.0).astype(i32) - 2

    def sel(vals, dtype):
        return jnp.where(
            lvl == 0, jnp.asarray(vals[0], dtype),
            jnp.where(lvl == 1, jnp.asarray(vals[1], dtype),
                      jnp.where(lvl == 2, jnp.asarray(vals[2], dtype),
                                jnp.asarray(vals[3], dtype))))

    scale = sel(_SCALES, f32)
    wl = sel(_WLS, i32)
    base = sel(_BASES, i32)

    roi_row = pid * _BR + lax.broadcasted_iota(i32, (_BR, 1), 0)
    bimg = roi_row // 512
    base = base + bimg * (wl * wl)

    x1s = x1 * scale
    y1s = y1 * scale
    roi_w = jnp.maximum(x2 * scale - x1s, 1.0)
    roi_h = jnp.maximum(y2 * scale - y1s, 1.0)
    bin_w = roi_w / float(_RES)
    bin_h = roi_h / float(_RES)

    s = lax.broadcasted_iota(i32, (1, _SLOTS), 1)
    p = s >> 4
    l = s & 15
    dy = (l >> 3) & 1
    dx = (l >> 2) & 1
    a = (l >> 1) & 1
    b = l & 1
    py = p // _RES
    px = p % _RES
    ky = 2 * py + dy
    kx = 2 * px + dx
    ty = (ky.astype(f32) + 0.5) * 0.5
    tx = (kx.astype(f32) + 0.5) * 0.5

    ys = y1s + ty * bin_h
    xs = x1s + tx * bin_w
    lf = wl.astype(f32)

    def interp(c):
        valid = (c >= -1.0) & (c <= lf)
        cc = jnp.maximum(c, 0.0)
        lo = jnp.minimum(jnp.floor(cc).astype(i32), wl - 1)
        hi = jnp.minimum(lo + 1, wl - 1)
        cc = jnp.where(lo >= wl - 1, lo.astype(f32), cc)
        frac = cc - lo.astype(f32)
        return lo, hi, frac, valid

    ylo, yhi, fy, vy = interp(ys)
    xlo, xhi, fx, vx = interp(xs)

    yc = jnp.where(a == 1, yhi, ylo)
    wy = jnp.where(a == 1, fy, 1.0 - fy)
    xc = jnp.where(b == 1, xhi, xlo)
    wx = jnp.where(b == 1, fx, 1.0 - fx)

    w = 0.25 * wy * wx
    w = jnp.where(vy & vx, w, 0.0)
    idx = base + yc * wl + xc

    idx_ref[...] = idx
    w_ref[...] = w


_prep = pl.pallas_call(
    _prep_body,
    grid=(_NROI // _BR,),
    in_specs=[pl.BlockSpec((_BR, 1), lambda i: (i, 0))] * 4,
    out_specs=[pl.BlockSpec((_BR, _SLOTS), lambda i: (i, 0))] * 2,
    out_shape=[
        jax.ShapeDtypeStruct((_NROI, _SLOTS), jnp.int32),
        jax.ShapeDtypeStruct((_NROI, _SLOTS), jnp.float32),
    ],
)


def _make_sc_pool(num_workers, ncores):
    rpw = _NROI // num_workers
    mesh = plsc.VectorSubcoreMesh(core_axis_name="c", subcore_axis_name="s")

    @functools.partial(
        pl.kernel,
        mesh=mesh,
        out_type=jax.ShapeDtypeStruct((_NROI, _C * _NPIX), jnp.float32),
        scratch_types=[
            pltpu.VMEM((_SLOTS,), jnp.int32),
            pltpu.VMEM((_SLOTS,), jnp.int32),
            pltpu.VMEM((_SLOTS,), jnp.float32),
            pltpu.VMEM((_SLOTS,), jnp.float32),
            pltpu.VMEM((2, _CHUNK, _C // 2), jnp.int32),
            pltpu.VMEM((_C * _NPIX,), jnp.float32),
            pltpu.SemaphoreType.DMA,
            pltpu.SemaphoreType.DMA,
            pltpu.SemaphoreType.DMA,
            pltpu.SemaphoreType.DMA,
        ],
    )
    def sc_pool(table, idx_all, w_all, out,
                idx_v0, idx_v1, w_v0, w_v1, buf, stage,
                sem0, sem1, isem0, isem1):
        wid = lax.axis_index("s") * ncores + lax.axis_index("c")
        sems = (sem0, sem1)
        isems = (isem0, isem1)
        idxs = (idx_v0, idx_v1)
        ws = (w_v0, w_v1)

        def start_chunk(il, c, slot):
            pltpu.async_copy(
                table.at[idxs[il].at[pl.ds(c * _CHUNK, _CHUNK)]],
                buf.at[slot], sems[slot])

        def chunk_desc(il, c, slot):
            return pltpu.make_async_copy(
                table.at[idxs[il].at[pl.ds(c * _CHUNK, _CHUNK)]],
                buf.at[slot], sems[slot])

        def start_meta(r, il):
            pltpu.async_copy(idx_all.at[r], idxs[il], isems[il])
            pltpu.async_copy(w_all.at[r], ws[il], isems[il])

        def wait_meta(r, il):
            pltpu.make_async_copy(idx_all.at[r], idxs[il], isems[il]).wait()
            pltpu.make_async_copy(w_all.at[r], ws[il], isems[il]).wait()

        def do_roi(i, r, il):
            def chunk_body(c, _c):
                slot = (i + c) & 1

                @pl.when(slot == 0)
                def _():
                    chunk_desc(il, c, 0).wait()
                    @pl.when(c + 1 < _NCHUNK)
                    def _():
                        start_chunk(il, c + 1, 1)

                @pl.when(slot == 1)
                def _():
                    chunk_desc(il, c, 1).wait()
                    @pl.when(c + 1 < _NCHUNK)
                    def _():
                        start_chunk(il, c + 1, 0)

                @pl.when((c + 1 == _NCHUNK) & (i + 1 < rpw))
                def _():
                    wait_meta(r + 1, 1 - il)
                    start_chunk(1 - il, 0, 1 - il)

                def pix_body(q, _q):
                    pglob = c * _CHUNK_PIX + q
                    wv = ws[il][pl.ds(pglob * 16, 16)]

                    def k_body(k, acc):
                        wk = lax.gather(
                            wv, jnp.full((16, 1), k, jnp.int32),
                            lax.GatherDimensionNumbers(
                                offset_dims=(), collapsed_slice_dims=(0,),
                                start_index_map=(0,)),
                            (1,),
                            mode=lax.GatherScatterMode.PROMISE_IN_BOUNDS)
                        row = q * 16 + k
                        new = []
                        for t in range(8):
                            packed = buf[slot, row, pl.ds(t * 16, 16)]
                            va = lax.bitcast_convert_type(
                                packed << 16, jnp.float32)
                            vb = lax.bitcast_convert_type(
                                (packed >> 16) << 16, jnp.float32)
                            new.append(acc[2 * t] + wk * va)
                            new.append(acc[2 * t + 1] + wk * vb)
                        return tuple(new)

                    zero = jnp.zeros((16,), jnp.float32)
                    acc = lax.fori_loop(0, 16, k_body, (zero,) * 16)
                    for j in range(16):
                        stage[pl.ds(pglob * _C + j * 16, 16)] = acc[j]
                    return 0

                lax.fori_loop(0, _CHUNK_PIX, pix_body, 0)
                return 0

            lax.fori_loop(0, _NCHUNK, chunk_body, 0)
            @pl.when(i + 2 < rpw)
            def _():
                start_meta(r + 2, il)

        r0 = wid * rpw
        start_meta(r0, 0)
        wait_meta(r0, 0)
        start_chunk(0, 0, 0)
        start_meta(r0 + 1, 1)

        def roi_body(i, _):
            r = wid * rpw + i

            @pl.when((i & 1) == 0)
            def _():
                do_roi(i, r, 0)

            @pl.when((i & 1) == 1)
            def _():
                do_roi(i, r, 1)

            pltpu.sync_copy(stage, out.at[r])
            return 0

        lax.fori_loop(0, rpw, roi_body, 0)

    return sc_pool


def kernel(feat0, feat1, feat2, feat3, feat4, proposals0, proposals1):
    del feat4
    feats = (feat0, feat1, feat2, feat3)
    rows = [jnp.transpose(f, (0, 2, 3, 1)).reshape(-1, _C) for f in feats]
    table = jnp.concatenate(rows, axis=0)
    table = lax.bitcast_convert_type(
        table.astype(jnp.bfloat16).reshape(_TROWS, _C // 2, 2), jnp.int32)

    props = jnp.concatenate([proposals0, proposals1], axis=0)
    cols = [props[:, k:k + 1] for k in range(4)]
    idx_all, w_all = _prep(*cols)

    info = plsc.get_sparse_core_info()
    nw = info.num_cores * info.num_subcores
    pooled = _make_sc_pool(nw, info.num_cores)(table, idx_all, w_all)
    pooled = pooled.reshape(_NROI, _NPIX, _C // 32, 2, 16)
    pooled = pooled.transpose(0, 2, 4, 3, 1)
    return pooled.reshape(_NROI, _C, _RES, _RES)


# speedup vs baseline: 1.0180x; 1.0180x over previous
import functools

import jax
import jax.numpy as jnp
from jax import lax
from jax.experimental import pallas as pl
from jax.experimental.pallas import tpu as pltpu
from jax.experimental.pallas import tpu_sc as plsc

_RES = 7
_NPIX = _RES * _RES
_SLOTS = _NPIX * 16
_CHUNK_PIX = 7
_CHUNK = _CHUNK_PIX * 16
_NCHUNK = _NPIX // _CHUNK_PIX
_SLOTS_PAD = 896
_C = 256
_NROI = 1024
_BR = 128

_WLS = (256, 128, 64, 32)
_BASES = (0, 2 * 256 * 256, 2 * 256 * 256 + 2 * 128 * 128,
          2 * 256 * 256 + 2 * 128 * 128 + 2 * 64 * 64)
_TROWS = _BASES[3] + 2 * 32 * 32
_SCALES = (0.25, 0.125, 0.0625, 0.03125)


def _prep_body(x1_ref, y1_ref, x2_ref, y2_ref, idx_ref, w_ref):
    i32 = jnp.int32
    f32 = jnp.float32
    pid = pl.program_id(0)
    x1 = x1_ref[...]
    y1 = y1_ref[...]
    x2 = x2_ref[...]
    y2 = y2_ref[...]

    area = (x2 - x1 + 1.0) * (y2 - y1 + 1.0)
    size = jnp.sqrt(area)
    lvlf = jnp.floor(4.0 + jnp.log2(size / 224.0 + 1e-6))
    lvl = jnp.clip(lvlf, 2.0, 5.0).astype(i32) - 2

    def sel(vals, dtype):
        return jnp.where(
            lvl == 0, jnp.asarray(vals[0], dtype),
            jnp.where(lvl == 1, jnp.asarray(vals[1], dtype),
                      jnp.where(lvl == 2, jnp.asarray(vals[2], dtype),
                                jnp.asarray(vals[3], dtype))))

    scale = sel(_SCALES, f32)
    wl = sel(_WLS, i32)
    base = sel(_BASES, i32)

    roi_row = pid * _BR + lax.broadcasted_iota(i32, (_BR, 1), 0)
    bimg = roi_row // 512
    base = base + bimg * (wl * wl)

    x1s = x1 * scale
    y1s = y1 * scale
    roi_w = jnp.maximum(x2 * scale - x1s, 1.0)
    roi_h = jnp.maximum(y2 * scale - y1s, 1.0)
    bin_w = roi_w / float(_RES)
    bin_h = roi_h / float(_RES)

    s = lax.broadcasted_iota(i32, (1, _SLOTS), 1)
    p = s >> 4
    l = s & 15
    dy = (l >> 3) & 1
    dx = (l >> 2) & 1
    a = (l >> 1) & 1
    b = l & 1
    py = p // _RES
    px = p % _RES
    ky = 2 * py + dy
    kx = 2 * px + dx
    ty = (ky.astype(f32) + 0.5) * 0.5
    tx = (kx.astype(f32) + 0.5) * 0.5

    ys = y1s + ty * bin_h
    xs = x1s + tx * bin_w
    lf = wl.astype(f32)

    def interp(c):
        valid = (c >= -1.0) & (c <= lf)
        cc = jnp.maximum(c, 0.0)
        lo = jnp.minimum(jnp.floor(cc).astype(i32), wl - 1)
        hi = jnp.minimum(lo + 1, wl - 1)
        cc = jnp.where(lo >= wl - 1, lo.astype(f32), cc)
        frac = cc - lo.astype(f32)
        return lo, hi, frac, valid

    ylo, yhi, fy, vy = interp(ys)
    xlo, xhi, fx, vx = interp(xs)

    yc = jnp.where(a == 1, yhi, ylo)
    wy = jnp.where(a == 1, fy, 1.0 - fy)
    xc = jnp.where(b == 1, xhi, xlo)
    wx = jnp.where(b == 1, fx, 1.0 - fx)

    w = 0.25 * wy * wx
    w = jnp.where(vy & vx, w, 0.0)
    idx = base + yc * wl + xc

    idx_ref[...] = idx
    w_ref[...] = w


_prep = pl.pallas_call(
    _prep_body,
    grid=(_NROI // _BR,),
    in_specs=[pl.BlockSpec((_BR, 1), lambda i: (i, 0))] * 4,
    out_specs=[pl.BlockSpec((_BR, _SLOTS), lambda i: (i, 0))] * 2,
    out_shape=[
        jax.ShapeDtypeStruct((_NROI, _SLOTS), jnp.int32),
        jax.ShapeDtypeStruct((_NROI, _SLOTS), jnp.float32),
    ],
)


def _make_sc_pool(num_workers, ncores):
    rpw = _NROI // num_workers
    mesh = plsc.VectorSubcoreMesh(core_axis_name="c", subcore_axis_name="s")

    @functools.partial(
        pl.kernel,
        mesh=mesh,
        out_type=jax.ShapeDtypeStruct((_NROI, _C * _NPIX), jnp.float32),
        scratch_types=[
            pltpu.VMEM((_SLOTS,), jnp.int32),
            pltpu.VMEM((_SLOTS,), jnp.int32),
            pltpu.VMEM((_SLOTS,), jnp.float32),
            pltpu.VMEM((_SLOTS,), jnp.float32),
            pltpu.VMEM((2, _CHUNK, _C // 2), jnp.int32),
            pltpu.VMEM((_C * _NPIX,), jnp.float32),
            pltpu.SemaphoreType.DMA,
            pltpu.SemaphoreType.DMA,
            pltpu.SemaphoreType.DMA,
            pltpu.SemaphoreType.DMA,
        ],
    )
    def sc_pool(table, idx_all, w_all, out,
                idx_v0, idx_v1, w_v0, w_v1, buf, stage,
                sem0, sem1, isem0, isem1):
        wid = lax.axis_index("s") * ncores + lax.axis_index("c")
        sems = (sem0, sem1)
        isems = (isem0, isem1)
        idxs = (idx_v0, idx_v1)
        ws = (w_v0, w_v1)

        def start_chunk(il, c, slot):
            pltpu.async_copy(
                table.at[idxs[il].at[pl.ds(c * _CHUNK, _CHUNK)]],
                buf.at[slot], sems[slot])

        def chunk_desc(il, c, slot):
            return pltpu.make_async_copy(
                table.at[idxs[il].at[pl.ds(c * _CHUNK, _CHUNK)]],
                buf.at[slot], sems[slot])

        def start_meta(r, il):
            pltpu.async_copy(idx_all.at[r], idxs[il], isems[il])
            pltpu.async_copy(w_all.at[r], ws[il], isems[il])

        def wait_meta(r, il):
            pltpu.make_async_copy(idx_all.at[r], idxs[il], isems[il]).wait()
            pltpu.make_async_copy(w_all.at[r], ws[il], isems[il]).wait()

        def do_roi(i, r, il):
            def chunk_body(c, _c):
                slot = (i + c) & 1

                @pl.when(slot == 0)
                def _():
                    chunk_desc(il, c, 0).wait()
                    @pl.when(c + 1 < _NCHUNK)
                    def _():
                        start_chunk(il, c + 1, 1)

                @pl.when(slot == 1)
                def _():
                    chunk_desc(il, c, 1).wait()
                    @pl.when(c + 1 < _NCHUNK)
                    def _():
                        start_chunk(il, c + 1, 0)

                @pl.when((c + 1 == _NCHUNK) & (i + 1 < rpw))
                def _():
                    wait_meta(r + 1, 1 - il)
                    start_chunk(1 - il, 0, 1 - il)

                def pix_body(q, _q):
                    pglob = c * _CHUNK_PIX + q
                    wv = ws[il][pl.ds(pglob * 16, 16)]

                    def k_body(k, acc):
                        wk = lax.gather(
                            wv, jnp.full((16, 1), k, jnp.int32),
                            lax.GatherDimensionNumbers(
                                offset_dims=(), collapsed_slice_dims=(0,),
                                start_index_map=(0,)),
                            (1,),
                            mode=lax.GatherScatterMode.PROMISE_IN_BOUNDS)
                        row = q * 16 + k
                        new = []
                        for t in range(8):
                            packed = buf[slot, row, pl.ds(t * 16, 16)]
                            va = lax.bitcast_convert_type(
                                packed << 16, jnp.float32)
                            vb = lax.bitcast_convert_type(
                                (packed >> 16) << 16, jnp.float32)
                            new.append(acc[2 * t] + wk * va)
                            new.append(acc[2 * t + 1] + wk * vb)
                        return tuple(new)

                    zero = jnp.zeros((16,), jnp.float32)
                    acc = lax.fori_loop(0, 16, k_body, (zero,) * 16)
                    for j in range(16):
                        stage[pl.ds(pglob * _C + j * 16, 16)] = acc[j]
                    return 0

                lax.fori_loop(0, _CHUNK_PIX, pix_body, 0)
                return 0

            lax.fori_loop(0, _NCHUNK, chunk_body, 0)
            @pl.when(i + 2 < rpw)
            def _():
                start_meta(r + 2, il)

        r0 = wid * rpw
        start_meta(r0, 0)
        wait_meta(r0, 0)
        start_chunk(0, 0, 0)
        start_meta(r0 + 1, 1)

        def roi_body(i, _):
            r = wid * rpw + i

            @pl.when((i & 1) == 0)
            def _():
                do_roi(i, r, 0)

            @pl.when((i & 1) == 1)
            def _():
                do_roi(i, r, 1)

            pltpu.sync_copy(stage, out.at[r])
            return 0

        lax.fori_loop(0, rpw, roi_body, 0)

    return sc_pool


def kernel(feat0, feat1, feat2, feat3, feat4, proposals0, proposals1):
    del feat4
    feats = (feat0, feat1, feat2, feat3)
    rows = [jnp.transpose(f.astype(jnp.bfloat16), (0, 2, 3, 1)).reshape(-1, _C)
            for f in feats]
    table = jnp.concatenate(rows, axis=0)
    table = lax.bitcast_convert_type(
        table.reshape(_TROWS, _C // 2, 2), jnp.int32)

    props = jnp.concatenate([proposals0, proposals1], axis=0)
    cols = [props[:, k:k + 1] for k in range(4)]
    idx_all, w_all = _prep(*cols)

    info = plsc.get_sparse_core_info()
    nw = info.num_cores * info.num_subcores
    pooled = _make_sc_pool(nw, info.num_cores)(table, idx_all, w_all)
    pooled = pooled.reshape(_NROI, _NPIX, _C // 32, 2, 16)
    pooled = pooled.transpose(0, 2, 4, 3, 1)
    return pooled.reshape(_NROI, _C, _RES, _RES)


# speedup vs baseline: 1.2573x; 1.2351x over previous
import functools

import jax
import jax.numpy as jnp
from jax import lax
from jax.experimental import pallas as pl
from jax.experimental.pallas import tpu as pltpu
from jax.experimental.pallas import tpu_sc as plsc

_RES = 7
_NPIX = _RES * _RES
_SLOTS = _NPIX * 16
_CHUNK_PIX = 7
_CHUNK = _CHUNK_PIX * 16
_NCHUNK = _NPIX // _CHUNK_PIX
_SLOTS_PAD = 896
_C = 256
_NROI = 1024
_BR = 128

_WLS = (256, 128, 64, 32)
_BASES = (0, 2 * 256 * 256, 2 * 256 * 256 + 2 * 128 * 128,
          2 * 256 * 256 + 2 * 128 * 128 + 2 * 64 * 64)
_TROWS = _BASES[3] + 2 * 32 * 32
_SCALES = (0.25, 0.125, 0.0625, 0.03125)


def _prep_body(x1_ref, y1_ref, x2_ref, y2_ref, idx_ref, w_ref):
    i32 = jnp.int32
    f32 = jnp.float32
    pid = pl.program_id(0)
    x1 = x1_ref[...]
    y1 = y1_ref[...]
    x2 = x2_ref[...]
    y2 = y2_ref[...]

    area = (x2 - x1 + 1.0) * (y2 - y1 + 1.0)
    size = jnp.sqrt(area)
    lvlf = jnp.floor(4.0 + jnp.log2(size / 224.0 + 1e-6))
    lvl = jnp.clip(lvlf, 2.0, 5.0).astype(i32) - 2

    def sel(vals, dtype):
        return jnp.where(
            lvl == 0, jnp.asarray(vals[0], dtype),
            jnp.where(lvl == 1, jnp.asarray(vals[1], dtype),
                      jnp.where(lvl == 2, jnp.asarray(vals[2], dtype),
                                jnp.asarray(vals[3], dtype))))

    scale = sel(_SCALES, f32)
    wl = sel(_WLS, i32)
    base = sel(_BASES, i32)

    roi_row = pid * _BR + lax.broadcasted_iota(i32, (_BR, 1), 0)
    bimg = roi_row // 512
    base = base + bimg * (wl * wl)

    x1s = x1 * scale
    y1s = y1 * scale
    roi_w = jnp.maximum(x2 * scale - x1s, 1.0)
    roi_h = jnp.maximum(y2 * scale - y1s, 1.0)
    bin_w = roi_w / float(_RES)
    bin_h = roi_h / float(_RES)

    s = lax.broadcasted_iota(i32, (1, _SLOTS), 1)
    p = s >> 4
    l = s & 15
    dy = (l >> 3) & 1
    dx = (l >> 2) & 1
    a = (l >> 1) & 1
    b = l & 1
    py = p // _RES
    px = p % _RES
    ky = 2 * py + dy
    kx = 2 * px + dx
    ty = (ky.astype(f32) + 0.5) * 0.5
    tx = (kx.astype(f32) + 0.5) * 0.5

    ys = y1s + ty * bin_h
    xs = x1s + tx * bin_w
    lf = wl.astype(f32)

    def interp(c):
        valid = (c >= -1.0) & (c <= lf)
        cc = jnp.maximum(c, 0.0)
        lo = jnp.minimum(jnp.floor(cc).astype(i32), wl - 1)
        hi = jnp.minimum(lo + 1, wl - 1)
        cc = jnp.where(lo >= wl - 1, lo.astype(f32), cc)
        frac = cc - lo.astype(f32)
        return lo, hi, frac, valid

    ylo, yhi, fy, vy = interp(ys)
    xlo, xhi, fx, vx = interp(xs)

    yc = jnp.where(a == 1, yhi, ylo)
    wy = jnp.where(a == 1, fy, 1.0 - fy)
    xc = jnp.where(b == 1, xhi, xlo)
    wx = jnp.where(b == 1, fx, 1.0 - fx)

    w = 0.25 * wy * wx
    w = jnp.where(vy & vx, w, 0.0)
    idx = base + yc * wl + xc

    idx_ref[...] = idx
    w_ref[...] = w


_prep = pl.pallas_call(
    _prep_body,
    grid=(_NROI // _BR,),
    in_specs=[pl.BlockSpec((_BR, 1), lambda i: (i, 0))] * 4,
    out_specs=[pl.BlockSpec((_BR, _SLOTS), lambda i: (i, 0))] * 2,
    out_shape=[
        jax.ShapeDtypeStruct((_NROI, _SLOTS), jnp.int32),
        jax.ShapeDtypeStruct((_NROI, _SLOTS), jnp.float32),
    ],
)


def _make_sc_pool(num_workers, ncores):
    rpw = _NROI // num_workers
    mesh = plsc.VectorSubcoreMesh(core_axis_name="c", subcore_axis_name="s")

    @functools.partial(
        pl.kernel,
        mesh=mesh,
        out_type=jax.ShapeDtypeStruct((_NROI, _C * _NPIX), jnp.float32),
        scratch_types=[
            pltpu.VMEM((_SLOTS,), jnp.int32),
            pltpu.VMEM((_SLOTS,), jnp.int32),
            pltpu.VMEM((_SLOTS,), jnp.float32),
            pltpu.VMEM((_SLOTS,), jnp.float32),
            pltpu.VMEM((2, _CHUNK, _C // 2), jnp.int32),
            pltpu.VMEM((_C * _NPIX,), jnp.float32),
            pltpu.SemaphoreType.DMA,
            pltpu.SemaphoreType.DMA,
            pltpu.SemaphoreType.DMA,
            pltpu.SemaphoreType.DMA,
        ],
    )
    def sc_pool(table, idx_all, w_all, out,
                idx_v0, idx_v1, w_v0, w_v1, buf, stage,
                sem0, sem1, isem0, isem1):
        wid = lax.axis_index("s") * ncores + lax.axis_index("c")
        sems = (sem0, sem1)
        isems = (isem0, isem1)
        idxs = (idx_v0, idx_v1)
        ws = (w_v0, w_v1)

        def start_chunk(il, c, slot):
            pltpu.async_copy(
                table.at[idxs[il].at[pl.ds(c * _CHUNK, _CHUNK)]],
                buf.at[slot], sems[slot])

        def chunk_desc(il, c, slot):
            return pltpu.make_async_copy(
                table.at[idxs[il].at[pl.ds(c * _CHUNK, _CHUNK)]],
                buf.at[slot], sems[slot])

        def start_meta(r, il):
            pltpu.async_copy(idx_all.at[r], idxs[il], isems[il])
            pltpu.async_copy(w_all.at[r], ws[il], isems[il])

        def wait_meta(r, il):
            pltpu.make_async_copy(idx_all.at[r], idxs[il], isems[il]).wait()
            pltpu.make_async_copy(w_all.at[r], ws[il], isems[il]).wait()

        def do_roi(i, r, il):
            def chunk_body(c, _c):
                slot = (i + c) & 1

                @pl.when(slot == 0)
                def _():
                    chunk_desc(il, c, 0).wait()
                    @pl.when(c + 1 < _NCHUNK)
                    def _():
                        start_chunk(il, c + 1, 1)

                @pl.when(slot == 1)
                def _():
                    chunk_desc(il, c, 1).wait()
                    @pl.when(c + 1 < _NCHUNK)
                    def _():
                        start_chunk(il, c + 1, 0)

                @pl.when((c + 1 == _NCHUNK) & (i + 1 < rpw))
                def _():
                    wait_meta(r + 1, 1 - il)
                    start_chunk(1 - il, 0, 1 - il)

                def pix_body(q, _q):
                    pglob = c * _CHUNK_PIX + q
                    wv = ws[il][pl.ds(pglob * 16, 16)]

                    def k_body(k, acc):
                        wk = lax.gather(
                            wv, jnp.full((16, 1), k, jnp.int32),
                            lax.GatherDimensionNumbers(
                                offset_dims=(), collapsed_slice_dims=(0,),
                                start_index_map=(0,)),
                            (1,),
                            mode=lax.GatherScatterMode.PROMISE_IN_BOUNDS)
                        row = q * 16 + k
                        new = []
                        for t in range(8):
                            packed = buf[slot, row, pl.ds(t * 16, 16)]
                            va = lax.bitcast_convert_type(
                                packed << 16, jnp.float32)
                            vb = lax.bitcast_convert_type(
                                (packed >> 16) << 16, jnp.float32)
                            new.append(acc[2 * t] + wk * va)
                            new.append(acc[2 * t + 1] + wk * vb)
                        return tuple(new)

                    zero = jnp.zeros((16,), jnp.float32)
                    acc = lax.fori_loop(0, 16, k_body, (zero,) * 16)
                    for j in range(16):
                        stage[pl.ds(pglob * _C + j * 16, 16)] = acc[j]
                    return 0

                lax.fori_loop(0, _CHUNK_PIX, pix_body, 0)
                return 0

            lax.fori_loop(0, _NCHUNK, chunk_body, 0)
            @pl.when(i + 2 < rpw)
            def _():
                start_meta(r + 2, il)

        r0 = wid * rpw
        start_meta(r0, 0)
        wait_meta(r0, 0)
        start_chunk(0, 0, 0)
        start_meta(r0 + 1, 1)

        def roi_body(i, _):
            r = wid * rpw + i

            @pl.when((i & 1) == 0)
            def _():
                do_roi(i, r, 0)

            @pl.when((i & 1) == 1)
            def _():
                do_roi(i, r, 1)

            pltpu.sync_copy(stage, out.at[r])
            return 0

        lax.fori_loop(0, rpw, roi_body, 0)

    return sc_pool


def kernel(feat0, feat1, feat2, feat3, feat4, proposals0, proposals1):
    del feat4
    feats = (feat0, feat1, feat2, feat3)

    def pack_feat(f):
        u = lax.bitcast_convert_type(f, jnp.uint32)
        r = (u + 0x7FFF + ((u >> 16) & 1)) >> 16
        word = r[:, 0::2] | (r[:, 1::2] << 16)
        word = lax.bitcast_convert_type(word, jnp.int32)
        return jnp.transpose(word, (0, 2, 3, 1)).reshape(-1, _C // 2)

    table = jnp.concatenate([pack_feat(f) for f in feats], axis=0)

    props = jnp.concatenate([proposals0, proposals1], axis=0)
    cols = [props[:, k:k + 1] for k in range(4)]
    idx_all, w_all = _prep(*cols)

    info = plsc.get_sparse_core_info()
    nw = info.num_cores * info.num_subcores
    pooled = _make_sc_pool(nw, info.num_cores)(table, idx_all, w_all)
    pooled = pooled.reshape(_NROI, _NPIX, _C // 32, 2, 16)
    pooled = pooled.transpose(0, 2, 4, 3, 1)
    return pooled.reshape(_NROI, _C, _RES, _RES)


# speedup vs baseline: 1.9320x; 1.5366x over previous
import functools

import jax
import jax.numpy as jnp
from jax import lax
from jax.experimental import pallas as pl
from jax.experimental.pallas import tpu as pltpu
from jax.experimental.pallas import tpu_sc as plsc

_RES = 7
_NPIX = _RES * _RES
_SLOTS = _NPIX * 16
_CHUNK_PIX = 7
_CHUNK = _CHUNK_PIX * 16
_NCHUNK = _NPIX // _CHUNK_PIX
_SLOTS_PAD = 896
_C = 256
_NROI = 1024
_BR = 128

_WLS = (256, 128, 64, 32)
_BASES = (0, 2 * 256 * 256, 2 * 256 * 256 + 2 * 128 * 128,
          2 * 256 * 256 + 2 * 128 * 128 + 2 * 64 * 64)
_TROWS = _BASES[3] + 2 * 32 * 32
_SCALES = (0.25, 0.125, 0.0625, 0.03125)


def _prep_body(x1_ref, y1_ref, x2_ref, y2_ref, idx_ref, w_ref):
    i32 = jnp.int32
    f32 = jnp.float32
    pid = pl.program_id(0)
    x1 = x1_ref[...]
    y1 = y1_ref[...]
    x2 = x2_ref[...]
    y2 = y2_ref[...]

    area = (x2 - x1 + 1.0) * (y2 - y1 + 1.0)
    size = jnp.sqrt(area)
    lvlf = jnp.floor(4.0 + jnp.log2(size / 224.0 + 1e-6))
    lvl = jnp.clip(lvlf, 2.0, 5.0).astype(i32) - 2

    def sel(vals, dtype):
        return jnp.where(
            lvl == 0, jnp.asarray(vals[0], dtype),
            jnp.where(lvl == 1, jnp.asarray(vals[1], dtype),
                      jnp.where(lvl == 2, jnp.asarray(vals[2], dtype),
                                jnp.asarray(vals[3], dtype))))

    scale = sel(_SCALES, f32)
    wl = sel(_WLS, i32)
    base = sel(_BASES, i32)

    roi_row = pid * _BR + lax.broadcasted_iota(i32, (_BR, 1), 0)
    bimg = roi_row // 512
    base = base + bimg * (wl * wl)

    x1s = x1 * scale
    y1s = y1 * scale
    roi_w = jnp.maximum(x2 * scale - x1s, 1.0)
    roi_h = jnp.maximum(y2 * scale - y1s, 1.0)
    bin_w = roi_w / float(_RES)
    bin_h = roi_h / float(_RES)

    s = lax.broadcasted_iota(i32, (1, _SLOTS), 1)
    p = s >> 4
    l = s & 15
    dy = (l >> 3) & 1
    dx = (l >> 2) & 1
    a = (l >> 1) & 1
    b = l & 1
    py = p // _RES
    px = p % _RES
    ky = 2 * py + dy
    kx = 2 * px + dx
    ty = (ky.astype(f32) + 0.5) * 0.5
    tx = (kx.astype(f32) + 0.5) * 0.5

    ys = y1s + ty * bin_h
    xs = x1s + tx * bin_w
    lf = wl.astype(f32)

    def interp(c):
        valid = (c >= -1.0) & (c <= lf)
        cc = jnp.maximum(c, 0.0)
        lo = jnp.minimum(jnp.floor(cc).astype(i32), wl - 1)
        hi = jnp.minimum(lo + 1, wl - 1)
        cc = jnp.where(lo >= wl - 1, lo.astype(f32), cc)
        frac = cc - lo.astype(f32)
        return lo, hi, frac, valid

    ylo, yhi, fy, vy = interp(ys)
    xlo, xhi, fx, vx = interp(xs)

    yc = jnp.where(a == 1, yhi, ylo)
    wy = jnp.where(a == 1, fy, 1.0 - fy)
    xc = jnp.where(b == 1, xhi, xlo)
    wx = jnp.where(b == 1, fx, 1.0 - fx)

    w = 0.25 * wy * wx
    w = jnp.where(vy & vx, w, 0.0)
    idx = base + yc * wl + xc

    idx_ref[...] = idx
    w_ref[...] = w


_prep = pl.pallas_call(
    _prep_body,
    grid=(_NROI // _BR,),
    in_specs=[pl.BlockSpec((_BR, 1), lambda i: (i, 0))] * 4,
    out_specs=[pl.BlockSpec((_BR, _SLOTS), lambda i: (i, 0))] * 2,
    out_shape=[
        jax.ShapeDtypeStruct((_NROI, _SLOTS), jnp.int32),
        jax.ShapeDtypeStruct((_NROI, _SLOTS), jnp.float32),
    ],
)


def _make_sc_pool(num_workers, ncores):
    rpw = _NROI // num_workers
    mesh = plsc.VectorSubcoreMesh(core_axis_name="c", subcore_axis_name="s")

    @functools.partial(
        pl.kernel,
        mesh=mesh,
        out_type=jax.ShapeDtypeStruct((_NROI, _C * _NPIX), jnp.float32),
        scratch_types=[
            pltpu.VMEM((_SLOTS,), jnp.int32),
            pltpu.VMEM((_SLOTS,), jnp.int32),
            pltpu.VMEM((_SLOTS,), jnp.float32),
            pltpu.VMEM((_SLOTS,), jnp.float32),
            pltpu.VMEM((2, _CHUNK, _C // 2), jnp.int32),
            pltpu.VMEM((_C * _NPIX,), jnp.float32),
            pltpu.SemaphoreType.DMA,
            pltpu.SemaphoreType.DMA,
            pltpu.SemaphoreType.DMA,
            pltpu.SemaphoreType.DMA,
        ],
    )
    def sc_pool(table, idx_all, w_all, out,
                idx_v0, idx_v1, w_v0, w_v1, buf, stage,
                sem0, sem1, isem0, isem1):
        wid = lax.axis_index("s") * ncores + lax.axis_index("c")
        sems = (sem0, sem1)
        isems = (isem0, isem1)
        idxs = (idx_v0, idx_v1)
        ws = (w_v0, w_v1)

        def start_chunk(il, c, slot):
            pltpu.async_copy(
                table.at[idxs[il].at[pl.ds(c * _CHUNK, _CHUNK)]],
                buf.at[slot], sems[slot])

        def chunk_desc(il, c, slot):
            return pltpu.make_async_copy(
                table.at[idxs[il].at[pl.ds(c * _CHUNK, _CHUNK)]],
                buf.at[slot], sems[slot])

        def start_meta(r, il):
            pltpu.async_copy(idx_all.at[r], idxs[il], isems[il])
            pltpu.async_copy(w_all.at[r], ws[il], isems[il])

        def wait_meta(r, il):
            pltpu.make_async_copy(idx_all.at[r], idxs[il], isems[il]).wait()
            pltpu.make_async_copy(w_all.at[r], ws[il], isems[il]).wait()

        def do_roi(i, r, il):
            def chunk_body(c, _c):
                slot = (i + c) & 1

                @pl.when(slot == 0)
                def _():
                    chunk_desc(il, c, 0).wait()
                    @pl.when(c + 1 < _NCHUNK)
                    def _():
                        start_chunk(il, c + 1, 1)

                @pl.when(slot == 1)
                def _():
                    chunk_desc(il, c, 1).wait()
                    @pl.when(c + 1 < _NCHUNK)
                    def _():
                        start_chunk(il, c + 1, 0)

                @pl.when((c + 1 == _NCHUNK) & (i + 1 < rpw))
                def _():
                    wait_meta(r + 1, 1 - il)
                    start_chunk(1 - il, 0, 1 - il)

                def pix_body(q, _q):
                    pglob = c * _CHUNK_PIX + q
                    wv = ws[il][pl.ds(pglob * 16, 16)]

                    def k_body(k, acc):
                        wk = lax.gather(
                            wv, jnp.full((16, 1), k, jnp.int32),
                            lax.GatherDimensionNumbers(
                                offset_dims=(), collapsed_slice_dims=(0,),
                                start_index_map=(0,)),
                            (1,),
                            mode=lax.GatherScatterMode.PROMISE_IN_BOUNDS)
                        row = q * 16 + k
                        lo_new = []
                        hi_new = []
                        for t in range(8):
                            packed = buf[slot, row, pl.ds(t * 16, 16)]
                            va = lax.bitcast_convert_type(
                                packed << 16, jnp.float32)
                            vb = lax.bitcast_convert_type(
                                (packed >> 16) << 16, jnp.float32)
                            lo_new.append(acc[t] + wk * va)
                            hi_new.append(acc[8 + t] + wk * vb)
                        return tuple(lo_new + hi_new)

                    zero = jnp.zeros((16,), jnp.float32)
                    acc = lax.fori_loop(0, 16, k_body, (zero,) * 16)
                    for j in range(16):
                        stage[pl.ds(pglob * _C + j * 16, 16)] = acc[j]
                    return 0

                lax.fori_loop(0, _CHUNK_PIX, pix_body, 0)
                return 0

            lax.fori_loop(0, _NCHUNK, chunk_body, 0)
            @pl.when(i + 2 < rpw)
            def _():
                start_meta(r + 2, il)

        r0 = wid * rpw
        start_meta(r0, 0)
        wait_meta(r0, 0)
        start_chunk(0, 0, 0)
        start_meta(r0 + 1, 1)

        def roi_body(i, _):
            r = wid * rpw + i

            @pl.when((i & 1) == 0)
            def _():
                do_roi(i, r, 0)

            @pl.when((i & 1) == 1)
            def _():
                do_roi(i, r, 1)

            pltpu.sync_copy(stage, out.at[r])
            return 0

        lax.fori_loop(0, rpw, roi_body, 0)

    return sc_pool


def kernel(feat0, feat1, feat2, feat3, feat4, proposals0, proposals1):
    del feat4
    feats = (feat0, feat1, feat2, feat3)

    def pack_feat(f):
        u = lax.bitcast_convert_type(f, jnp.uint32)
        r = (u + 0x7FFF + ((u >> 16) & 1)) >> 16
        word = r[:, :_C // 2] | (r[:, _C // 2:] << 16)
        word = lax.bitcast_convert_type(word, jnp.int32)
        return jnp.transpose(word, (0, 2, 3, 1)).reshape(-1, _C // 2)

    table = jnp.concatenate([pack_feat(f) for f in feats], axis=0)

    props = jnp.concatenate([proposals0, proposals1], axis=0)
    cols = [props[:, k:k + 1] for k in range(4)]
    idx_all, w_all = _prep(*cols)

    info = plsc.get_sparse_core_info()
    nw = info.num_cores * info.num_subcores
    pooled = _make_sc_pool(nw, info.num_cores)(table, idx_all, w_all)
    pooled = pooled.reshape(_NROI, _NPIX, _C).transpose(0, 2, 1)
    return pooled.reshape(_NROI, _C, _RES, _RES)


# speedup vs baseline: 2.0604x; 1.0665x over previous
import functools

import jax
import jax.numpy as jnp
from jax import lax
from jax.experimental import pallas as pl
from jax.experimental.pallas import tpu as pltpu
from jax.experimental.pallas import tpu_sc as plsc

_RES = 7
_NPIX = _RES * _RES
_SLOTS = _NPIX * 16
_CHUNK_PIX = 7
_CHUNK = _CHUNK_PIX * 16
_NCHUNK = _NPIX // _CHUNK_PIX
_SLOTS_PAD = 896
_C = 256
_NROI = 1024
_BR = 128

_WLS = (256, 128, 64, 32)
_BASES = (0, 2 * 256 * 256, 2 * 256 * 256 + 2 * 128 * 128,
          2 * 256 * 256 + 2 * 128 * 128 + 2 * 64 * 64)
_TROWS = _BASES[3] + 2 * 32 * 32
_SCALES = (0.25, 0.125, 0.0625, 0.03125)


def _prep_body(x1_ref, y1_ref, x2_ref, y2_ref, idx_ref, w_ref):
    i32 = jnp.int32
    f32 = jnp.float32
    pid = pl.program_id(0)
    x1 = x1_ref[...]
    y1 = y1_ref[...]
    x2 = x2_ref[...]
    y2 = y2_ref[...]

    area = (x2 - x1 + 1.0) * (y2 - y1 + 1.0)
    size = jnp.sqrt(area)
    lvlf = jnp.floor(4.0 + jnp.log2(size / 224.0 + 1e-6))
    lvl = jnp.clip(lvlf, 2.0, 5.0).astype(i32) - 2

    def sel(vals, dtype):
        return jnp.where(
            lvl == 0, jnp.asarray(vals[0], dtype),
            jnp.where(lvl == 1, jnp.asarray(vals[1], dtype),
                      jnp.where(lvl == 2, jnp.asarray(vals[2], dtype),
                                jnp.asarray(vals[3], dtype))))

    scale = sel(_SCALES, f32)
    wl = sel(_WLS, i32)
    base = sel(_BASES, i32)

    roi_row = pid * _BR + lax.broadcasted_iota(i32, (_BR, 1), 0)
    bimg = roi_row // 512
    base = base + bimg * (wl * wl)

    x1s = x1 * scale
    y1s = y1 * scale
    roi_w = jnp.maximum(x2 * scale - x1s, 1.0)
    roi_h = jnp.maximum(y2 * scale - y1s, 1.0)
    bin_w = roi_w / float(_RES)
    bin_h = roi_h / float(_RES)

    s = lax.broadcasted_iota(i32, (1, _SLOTS), 1)
    p = s >> 4
    l = s & 15
    dy = (l >> 3) & 1
    dx = (l >> 2) & 1
    a = (l >> 1) & 1
    b = l & 1
    py = p // _RES
    px = p % _RES
    ky = 2 * py + dy
    kx = 2 * px + dx
    ty = (ky.astype(f32) + 0.5) * 0.5
    tx = (kx.astype(f32) + 0.5) * 0.5

    ys = y1s + ty * bin_h
    xs = x1s + tx * bin_w
    lf = wl.astype(f32)

    def interp(c):
        valid = (c >= -1.0) & (c <= lf)
        cc = jnp.maximum(c, 0.0)
        lo = jnp.minimum(jnp.floor(cc).astype(i32), wl - 1)
        hi = jnp.minimum(lo + 1, wl - 1)
        cc = jnp.where(lo >= wl - 1, lo.astype(f32), cc)
        frac = cc - lo.astype(f32)
        return lo, hi, frac, valid

    ylo, yhi, fy, vy = interp(ys)
    xlo, xhi, fx, vx = interp(xs)

    yc = jnp.where(a == 1, yhi, ylo)
    wy = jnp.where(a == 1, fy, 1.0 - fy)
    xc = jnp.where(b == 1, xhi, xlo)
    wx = jnp.where(b == 1, fx, 1.0 - fx)

    w = 0.25 * wy * wx
    w = jnp.where(vy & vx, w, 0.0)
    idx = base + yc * wl + xc

    idx_ref[...] = idx
    w_ref[...] = w


_prep = pl.pallas_call(
    _prep_body,
    grid=(_NROI // _BR,),
    in_specs=[pl.BlockSpec((_BR, 1), lambda i: (i, 0))] * 4,
    out_specs=[pl.BlockSpec((_BR, _SLOTS), lambda i: (i, 0))] * 2,
    out_shape=[
        jax.ShapeDtypeStruct((_NROI, _SLOTS), jnp.int32),
        jax.ShapeDtypeStruct((_NROI, _SLOTS), jnp.float32),
    ],
)


def _make_sc_pool(num_workers, ncores):
    rpw = _NROI // num_workers
    mesh = plsc.VectorSubcoreMesh(core_axis_name="c", subcore_axis_name="s")

    @functools.partial(
        pl.kernel,
        mesh=mesh,
        out_type=jax.ShapeDtypeStruct((_NROI, _C * _NPIX), jnp.float32),
        scratch_types=[
            pltpu.VMEM((_SLOTS,), jnp.int32),
            pltpu.VMEM((_SLOTS,), jnp.int32),
            pltpu.VMEM((_SLOTS,), jnp.float32),
            pltpu.VMEM((_SLOTS,), jnp.float32),
            pltpu.VMEM((3, _CHUNK, _C // 2), jnp.int32),
            pltpu.VMEM((_C * _NPIX,), jnp.float32),
            pltpu.SemaphoreType.DMA,
            pltpu.SemaphoreType.DMA,
            pltpu.SemaphoreType.DMA,
            pltpu.SemaphoreType.DMA,
            pltpu.SemaphoreType.DMA,
        ],
    )
    def sc_pool(table, idx_all, w_all, out,
                idx_v0, idx_v1, w_v0, w_v1, buf, stage,
                sem0, sem1, sem2, isem0, isem1):
        wid = lax.axis_index("s") * ncores + lax.axis_index("c")
        sems = (sem0, sem1, sem2)
        isems = (isem0, isem1)
        idxs = (idx_v0, idx_v1)
        ws = (w_v0, w_v1)

        def start_chunk(il, c, slot):
            pltpu.async_copy(
                table.at[idxs[il].at[pl.ds(c * _CHUNK, _CHUNK)]],
                buf.at[slot], sems[slot])

        def chunk_desc(il, c, slot):
            return pltpu.make_async_copy(
                table.at[idxs[il].at[pl.ds(c * _CHUNK, _CHUNK)]],
                buf.at[slot], sems[slot])

        def start_meta(r, il):
            pltpu.async_copy(idx_all.at[r], idxs[il], isems[il])
            pltpu.async_copy(w_all.at[r], ws[il], isems[il])

        def wait_meta(r, il):
            pltpu.make_async_copy(idx_all.at[r], idxs[il], isems[il]).wait()
            pltpu.make_async_copy(w_all.at[r], ws[il], isems[il]).wait()

        def do_roi(i, r, il):
            def chunk_body(c, _c):
                s = lax.rem(i + c, 3)

                def issue_next(t):
                    @pl.when(c < _NCHUNK - 2)
                    def _():
                        start_chunk(il, c + 2, t)

                    @pl.when((c == _NCHUNK - 2) & (i + 1 < rpw))
                    def _():
                        wait_meta(r + 1, 1 - il)
                        start_chunk(1 - il, 0, t)

                    @pl.when((c == _NCHUNK - 1) & (i + 1 < rpw))
                    def _():
                        start_chunk(1 - il, 1, t)

                for scur in range(3):
                    @pl.when(s == scur)
                    def _(scur=scur):
                        chunk_desc(il, c, scur).wait()
                        issue_next((scur + 2) % 3)

                def pix_body(q, _q):
                    pglob = c * _CHUNK_PIX + q
                    wv = ws[il][pl.ds(pglob * 16, 16)]

                    def k_body(k, acc):
                        wk = lax.gather(
                            wv, jnp.full((16, 1), k, jnp.int32),
                            lax.GatherDimensionNumbers(
                                offset_dims=(), collapsed_slice_dims=(0,),
                                start_index_map=(0,)),
                            (1,),
                            mode=lax.GatherScatterMode.PROMISE_IN_BOUNDS)
                        row = q * 16 + k
                        lo_new = []
                        hi_new = []
                        for t in range(8):
                            packed = buf[s, row, pl.ds(t * 16, 16)]
                            va = lax.bitcast_convert_type(
                                packed << 16, jnp.float32)
                            vb = lax.bitcast_convert_type(
                                (packed >> 16) << 16, jnp.float32)
                            lo_new.append(acc[t] + wk * va)
                            hi_new.append(acc[8 + t] + wk * vb)
                        return tuple(lo_new + hi_new)

                    zero = jnp.zeros((16,), jnp.float32)
                    acc = lax.fori_loop(0, 16, k_body, (zero,) * 16)
                    for j in range(16):
                        stage[pl.ds(pglob * _C + j * 16, 16)] = acc[j]
                    return 0

                lax.fori_loop(0, _CHUNK_PIX, pix_body, 0)
                return 0

            lax.fori_loop(0, _NCHUNK, chunk_body, 0)
            @pl.when(i + 2 < rpw)
            def _():
                start_meta(r + 2, il)

        r0 = wid * rpw
        start_meta(r0, 0)
        wait_meta(r0, 0)
        start_chunk(0, 0, 0)
        start_chunk(0, 1, 1)
        start_meta(r0 + 1, 1)

        def roi_body(i, _):
            r = wid * rpw + i

            @pl.when((i & 1) == 0)
            def _():
                do_roi(i, r, 0)

            @pl.when((i & 1) == 1)
            def _():
                do_roi(i, r, 1)

            pltpu.sync_copy(stage, out.at[r])
            return 0

        lax.fori_loop(0, rpw, roi_body, 0)

    return sc_pool


def kernel(feat0, feat1, feat2, feat3, feat4, proposals0, proposals1):
    del feat4
    feats = (feat0, feat1, feat2, feat3)

    def pack_feat(f):
        u = lax.bitcast_convert_type(f, jnp.uint32)
        r = (u + 0x7FFF + ((u >> 16) & 1)) >> 16
        word = r[:, :_C // 2] | (r[:, _C // 2:] << 16)
        word = lax.bitcast_convert_type(word, jnp.int32)
        return jnp.transpose(word, (0, 2, 3, 1)).reshape(-1, _C // 2)

    table = jnp.concatenate([pack_feat(f) for f in feats], axis=0)

    props = jnp.concatenate([proposals0, proposals1], axis=0)
    cols = [props[:, k:k + 1] for k in range(4)]
    idx_all, w_all = _prep(*cols)

    info = plsc.get_sparse_core_info()
    nw = info.num_cores * info.num_subcores
    pooled = _make_sc_pool(nw, info.num_cores)(table, idx_all, w_all)
    pooled = pooled.reshape(_NROI, _NPIX, _C).transpose(0, 2, 1)
    return pooled.reshape(_NROI, _C, _RES, _RES)
